# Initial kernel scaffold; baseline (speedup 1.0000x reference)
import jax
import jax.numpy as jnp
from jax.experimental import pallas as pl


def kernel(x, edge_index, W1, b1, W2, b2):
    raise NotImplementedError("write your pallas kernel here")


# speedup vs baseline: 9.0161x; 9.0161x over previous
import functools

import jax
import jax.numpy as jnp
from jax import lax
from jax.experimental import pallas as pl
from jax.experimental.pallas import tpu as pltpu
from jax.experimental.pallas import tpu_sc as plsc

N = 10000
C = 128
E = 320000
NC, NS = 2, 16
NW = NC * NS
CHUNK = 128
CPW = 80
E_PAD = NW * CPW * CHUNK
N_ACC = 10240
ZR = N_ACC // NS

@functools.lru_cache(maxsize=None)
def _sc_mesh():
    return plsc.VectorSubcoreMesh(
        core_axis_name="c", subcore_axis_name="s",
        num_cores=NC, num_subcores=NS)


HROWS = N_ACC // CHUNK


def _deg_body(dst_hbm, out_hbm, idxv, hist, rowidx, shared, sem):
    c = lax.axis_index("c")
    s = lax.axis_index("s")
    wid = c * NS + s
    pltpu.sync_copy(dst_hbm.at[pl.ds(wid * CPW, CPW)], idxv)
    ones16 = jnp.ones((16,), jnp.float32)
    zeros16 = jnp.zeros((16,), jnp.float32)
    iota16 = lax.iota(jnp.int32, 16)
    for k in range(HROWS // 16):
        rowidx[pl.ds(k * 16, 16)] = iota16 + k * 16

    def zbody(r, carry):
        for k in range(8):
            hist[r, pl.ds(k * 16, 16)] = zeros16
        return carry

    lax.fori_loop(0, HROWS, zbody, 0)

    @pl.when(s == 0)
    def _():
        pltpu.sync_copy(hist, shared)
    plsc.subcore_barrier()

    def body(r, carry):
        for k in range(8):
            idx = idxv[r, pl.ds(k * 16, 16)]
            plsc.addupdate_scatter(
                hist, [lax.shift_right_logical(idx, 7),
                       lax.bitwise_and(idx, 127)], ones16)
        return carry

    lax.fori_loop(0, CPW, body, 0)
    pltpu.sync_copy(hist, shared.at[rowidx], add=True)
    plsc.subcore_barrier()

    @pl.when(s == 0)
    def _():
        pltpu.sync_copy(shared, hist)
        pltpu.sync_copy(hist, out_hbm.at[pl.ds(c * HROWS, HROWS)])


@functools.lru_cache(maxsize=None)
def _deg_kernel():
    return pl.kernel(
        _deg_body,
        out_type=jax.ShapeDtypeStruct((NC * HROWS, CHUNK), jnp.float32),
        mesh=_sc_mesh(),
        compiler_params=pltpu.CompilerParams(needs_layout_passes=False),
        scratch_types=[
            pltpu.VMEM((CPW, CHUNK), jnp.int32),
            pltpu.VMEM((HROWS, CHUNK), jnp.float32),
            pltpu.VMEM((HROWS,), jnp.int32),
            pltpu.VMEM_SHARED((HROWS, CHUNK), jnp.float32),
            pltpu.SemaphoreType.DMA,
        ],
    )


def _scat_body(g_hbm, src_hbm, dst_hbm, zeros_hbm, out_hbm,
               srcv, dstv, bufA, acc, sem):
    c = lax.axis_index("c")
    s = lax.axis_index("s")
    wid = c * NS + s
    pltpu.sync_copy(src_hbm.at[pl.ds(wid * CPW, CPW)], srcv)
    pltpu.sync_copy(dst_hbm.at[pl.ds(wid * CPW, CPW)], dstv)
    pltpu.sync_copy(zeros_hbm, bufA)
    base = s * ZR
    for k in range(5):
        pltpu.sync_copy(bufA, acc.at[pl.ds(base + k * 128, 128)])
    plsc.subcore_barrier()

    def body(j, carry):
        pltpu.async_copy(g_hbm.at[srcv.at[j]], bufA, sem).wait()
        pltpu.sync_copy(bufA, acc.at[dstv.at[j]], add=True)
        return carry

    lax.fori_loop(0, CPW, body, 0)
    plsc.subcore_barrier()
    for k in range(5):
        pltpu.sync_copy(acc.at[pl.ds(base + k * 128, 128)], bufA)
        pltpu.sync_copy(bufA, out_hbm.at[pl.ds(c * N_ACC + base + k * 128, 128)])


@functools.lru_cache(maxsize=None)
def _scat_kernel():
    return pl.kernel(
        _scat_body,
        out_type=jax.ShapeDtypeStruct((NC * N_ACC, C), jnp.float32),
        mesh=_sc_mesh(),
        scratch_types=[
            pltpu.VMEM((CPW, CHUNK), jnp.int32),
            pltpu.VMEM((CPW, CHUNK), jnp.int32),
            pltpu.VMEM((CHUNK, C), jnp.float32),
            pltpu.VMEM_SHARED((N_ACC, C), jnp.float32),
            pltpu.SemaphoreType.DMA,
        ],
    )


BL = 2000


def _d_from_hist(hp_ref):
    deg = hp_ref[0] + hp_ref[1] + 1.0
    return lax.rsqrt(deg)


def _lin1_body(x_ref, w_ref, b_ref, hp_ref, g_ref, diag_ref):
    d = _d_from_hist(hp_ref)
    h = jnp.dot(x_ref[...], w_ref[...], preferred_element_type=jnp.float32)
    g_ref[...] = d * h
    diag_ref[...] = (d * d) * h + b_ref[...]


def _mid_body(p_ref, diag1_ref, w_ref, b_ref, hp_ref, g_ref, diag2_ref):
    d = _d_from_hist(hp_ref)
    out1 = jnp.maximum(d * (p_ref[0] + p_ref[1]) + diag1_ref[...], 0.0)
    h2 = jnp.dot(out1, w_ref[...], preferred_element_type=jnp.float32)
    g_ref[...] = d * h2
    diag2_ref[...] = (d * d) * h2 + b_ref[...]


def _fin_body(q_ref, diag2_ref, hp_ref, o_ref):
    d = _d_from_hist(hp_ref)
    o_ref[...] = d * (q_ref[0] + q_ref[1]) + diag2_ref[...]


def _row_spec():
    return pl.BlockSpec((BL, C), lambda i: (i, 0))


def _pair_spec():
    return pl.BlockSpec((2, BL, C), lambda i: (0, i, 0))


def _hist_spec():
    return pl.BlockSpec((2, BL, 1), lambda i: (0, i, 0))


def _const_spec(shape):
    return pl.BlockSpec(shape, lambda i: (0,) * len(shape))


_lin1 = pl.pallas_call(
    _lin1_body,
    grid=(N // BL,),
    in_specs=[_row_spec(), _const_spec((C, C)), _const_spec((1, C)),
              _hist_spec()],
    out_specs=[_row_spec(), _row_spec()],
    out_shape=[jax.ShapeDtypeStruct((N, C), jnp.float32)] * 2,
)

_mid = pl.pallas_call(
    _mid_body,
    grid=(N // BL,),
    in_specs=[_pair_spec(), _row_spec(), _const_spec((C, C)),
              _const_spec((1, C)), _hist_spec()],
    out_specs=[_row_spec(), _row_spec()],
    out_shape=[jax.ShapeDtypeStruct((N, C), jnp.float32)] * 2,
)

_fin = pl.pallas_call(
    _fin_body,
    grid=(N // BL,),
    in_specs=[_pair_spec(), _row_spec(), _hist_spec()],
    out_specs=_row_spec(),
    out_shape=jax.ShapeDtypeStruct((N, C), jnp.float32),
)


def kernel(x, edge_index, W1, b1, W2, b2):
    src = edge_index[0].astype(jnp.int32)
    dst = edge_index[1].astype(jnp.int32)
    pad = E_PAD - E
    src2d = jnp.concatenate([src, jnp.zeros((pad,), jnp.int32)]).reshape(
        NW * CPW, CHUNK)
    dst2d = jnp.concatenate([dst, jnp.full((pad,), N, jnp.int32)]).reshape(
        NW * CPW, CHUNK)
    zeros_c = jnp.zeros((CHUNK, C), jnp.float32)

    hp = _deg_kernel()(dst2d).reshape(NC, N_ACC)[:, :N, None]
    g1, diag1 = _lin1(x, W1, b1.reshape(1, C), hp)
    p = _scat_kernel()(g1, src2d, dst2d, zeros_c).reshape(NC, N_ACC, C)[:, :N]
    g2, diag2 = _mid(p, diag1, W2, b2.reshape(1, C), hp)
    q = _scat_kernel()(g2, src2d, dst2d, zeros_c).reshape(NC, N_ACC, C)[:, :N]
    return _fin(q, diag2, hp)


# speedup vs baseline: 9.6439x; 1.0696x over previous
import functools

import jax
import jax.numpy as jnp
from jax import lax
from jax.experimental import pallas as pl
from jax.experimental.pallas import tpu as pltpu
from jax.experimental.pallas import tpu_sc as plsc

N = 10000
C = 128
E = 320000
NC, NS = 2, 16
NW = NC * NS
CHUNK = 128
CPW = 80
E_PAD = NW * CPW * CHUNK
N_ACC = 10240
ZR = N_ACC // NS

@functools.lru_cache(maxsize=None)
def _sc_mesh():
    return plsc.VectorSubcoreMesh(
        core_axis_name="c", subcore_axis_name="s",
        num_cores=NC, num_subcores=NS)


HROWS = N_ACC // CHUNK


def _deg_body(dst_hbm, out_hbm, idxv, hist, rowidx, shared, sem):
    c = lax.axis_index("c")
    s = lax.axis_index("s")
    wid = c * NS + s
    pltpu.sync_copy(dst_hbm.at[pl.ds(wid * CPW, CPW)], idxv)
    ones16 = jnp.ones((16,), jnp.float32)
    zeros16 = jnp.zeros((16,), jnp.float32)
    iota16 = lax.iota(jnp.int32, 16)
    for k in range(HROWS // 16):
        rowidx[pl.ds(k * 16, 16)] = iota16 + k * 16

    def zbody(r, carry):
        for k in range(8):
            hist[r, pl.ds(k * 16, 16)] = zeros16
        return carry

    lax.fori_loop(0, HROWS, zbody, 0)

    @pl.when(s == 0)
    def _():
        pltpu.sync_copy(hist, shared)
    plsc.subcore_barrier()

    def body(r, carry):
        for k in range(8):
            idx = idxv[r, pl.ds(k * 16, 16)]
            plsc.addupdate_scatter(
                hist, [lax.shift_right_logical(idx, 7),
                       lax.bitwise_and(idx, 127)], ones16)
        return carry

    lax.fori_loop(0, CPW, body, 0)
    pltpu.sync_copy(hist, shared.at[rowidx], add=True)
    plsc.subcore_barrier()

    @pl.when(s == 0)
    def _():
        pltpu.sync_copy(shared, hist)
        pltpu.sync_copy(hist, out_hbm.at[pl.ds(c * HROWS, HROWS)])


@functools.lru_cache(maxsize=None)
def _deg_kernel():
    return pl.kernel(
        _deg_body,
        out_type=jax.ShapeDtypeStruct((NC * HROWS, CHUNK), jnp.float32),
        mesh=_sc_mesh(),
        compiler_params=pltpu.CompilerParams(needs_layout_passes=False),
        scratch_types=[
            pltpu.VMEM((CPW, CHUNK), jnp.int32),
            pltpu.VMEM((HROWS, CHUNK), jnp.float32),
            pltpu.VMEM((HROWS,), jnp.int32),
            pltpu.VMEM_SHARED((HROWS, CHUNK), jnp.float32),
            pltpu.SemaphoreType.DMA,
        ],
    )


GSZ = 8
NGI = CPW // GSZ


def _scat_body(g_hbm, src_hbm, dst_hbm, zeros_hbm, out_hbm,
               s0, s1, d0, d1, bufA, bufB, acc, gsA, gsB, is0, is1):
    sidx = [s0, s1]
    didx = [d0, d1]
    bufs = [bufA, bufB]
    gsems = [gsA, gsB]
    isems = [is0, is1]
    c = lax.axis_index("c")
    s = lax.axis_index("s")
    wid = c * NS + s
    row0 = wid * CPW
    pltpu.sync_copy(zeros_hbm, bufA)
    base = s * ZR
    for k in range(5):
        pltpu.sync_copy(bufA, acc.at[pl.ds(base + k * 128, 128)])
    plsc.subcore_barrier()

    pltpu.sync_copy(src_hbm.at[pl.ds(row0, GSZ)], sidx[0])
    pltpu.sync_copy(dst_hbm.at[pl.ds(row0, GSZ)], didx[0])
    pltpu.async_copy(g_hbm.at[sidx[0].at[0]], bufA, gsA)
    pltpu.async_copy(g_hbm.at[sidx[0].at[1]], bufB, gsB)
    pltpu.async_copy(src_hbm.at[pl.ds(row0 + GSZ, GSZ)], sidx[1], is1)
    pltpu.async_copy(dst_hbm.at[pl.ds(row0 + GSZ, GSZ)], didx[1], is1)

    def body(g, carry):
        p = lax.rem(g, 2)
        q = 1 - p
        for b in range(GSZ):
            buf = bufs[b % 2]
            gsem = gsems[b % 2]
            pltpu.make_async_copy(g_hbm.at[sidx[0].at[0]], buf, gsem).wait()
            for pp in range(2):
                @pl.when(p == pp)
                def _():
                    pltpu.sync_copy(buf, acc.at[didx[pp].at[b]], add=True)
            if b == GSZ - 3:
                @pl.when(g < NGI - 1)
                def _():
                    for pp in range(2):
                        @pl.when(q == pp)
                        def _():
                            pltpu.make_async_copy(
                                src_hbm.at[pl.ds(row0, GSZ)], sidx[pp],
                                isems[pp]).wait()
                            pltpu.make_async_copy(
                                dst_hbm.at[pl.ds(row0, GSZ)], didx[pp],
                                isems[pp]).wait()
            if b < GSZ - 2:
                for pp in range(2):
                    @pl.when(p == pp)
                    def _():
                        pltpu.async_copy(
                            g_hbm.at[sidx[pp].at[b + 2]], buf, gsem)
            else:
                @pl.when(g < NGI - 1)
                def _():
                    for pp in range(2):
                        @pl.when(q == pp)
                        def _():
                            pltpu.async_copy(
                                g_hbm.at[sidx[pp].at[b + 2 - GSZ]], buf, gsem)
        @pl.when(g < NGI - 2)
        def _():
            gr = row0 + (g + 2) * GSZ
            for pp in range(2):
                @pl.when(p == pp)
                def _():
                    pltpu.async_copy(src_hbm.at[pl.ds(gr, GSZ)], sidx[pp],
                                     isems[pp])
                    pltpu.async_copy(dst_hbm.at[pl.ds(gr, GSZ)], didx[pp],
                                     isems[pp])
        return carry

    lax.fori_loop(0, NGI, body, 0)
    plsc.subcore_barrier()
    for k in range(5):
        pltpu.sync_copy(acc.at[pl.ds(base + k * 128, 128)], bufA)
        pltpu.sync_copy(bufA,
                        out_hbm.at[pl.ds(c * N_ACC + base + k * 128, 128)])


@functools.lru_cache(maxsize=None)
def _scat_kernel():
    return pl.kernel(
        _scat_body,
        out_type=jax.ShapeDtypeStruct((NC * N_ACC, C), jnp.float32),
        mesh=_sc_mesh(),
        scratch_types=[
            pltpu.VMEM((GSZ, CHUNK), jnp.int32),
            pltpu.VMEM((GSZ, CHUNK), jnp.int32),
            pltpu.VMEM((GSZ, CHUNK), jnp.int32),
            pltpu.VMEM((GSZ, CHUNK), jnp.int32),
            pltpu.VMEM((CHUNK, C), jnp.float32),
            pltpu.VMEM((CHUNK, C), jnp.float32),
            pltpu.VMEM_SHARED((N_ACC, C), jnp.float32),
        ] + [pltpu.SemaphoreType.DMA] * 4,
    )


BL = 2000


def _d_from_hist(hp_ref):
    deg = hp_ref[0] + hp_ref[1] + 1.0
    return lax.rsqrt(deg)


def _lin1_body(x_ref, w_ref, b_ref, hp_ref, g_ref, diag_ref):
    d = _d_from_hist(hp_ref)
    h = jnp.dot(x_ref[...], w_ref[...], preferred_element_type=jnp.float32)
    g_ref[...] = d * h
    diag_ref[...] = (d * d) * h + b_ref[...]


def _mid_body(p_ref, diag1_ref, w_ref, b_ref, hp_ref, g_ref, diag2_ref):
    d = _d_from_hist(hp_ref)
    out1 = jnp.maximum(d * (p_ref[0] + p_ref[1]) + diag1_ref[...], 0.0)
    h2 = jnp.dot(out1, w_ref[...], preferred_element_type=jnp.float32)
    g_ref[...] = d * h2
    diag2_ref[...] = (d * d) * h2 + b_ref[...]


def _fin_body(q_ref, diag2_ref, hp_ref, o_ref):
    d = _d_from_hist(hp_ref)
    o_ref[...] = d * (q_ref[0] + q_ref[1]) + diag2_ref[...]


def _row_spec():
    return pl.BlockSpec((BL, C), lambda i: (i, 0))


def _pair_spec():
    return pl.BlockSpec((2, BL, C), lambda i: (0, i, 0))


def _hist_spec():
    return pl.BlockSpec((2, BL, 1), lambda i: (0, i, 0))


def _const_spec(shape):
    return pl.BlockSpec(shape, lambda i: (0,) * len(shape))


_lin1 = pl.pallas_call(
    _lin1_body,
    grid=(N // BL,),
    in_specs=[_row_spec(), _const_spec((C, C)), _const_spec((1, C)),
              _hist_spec()],
    out_specs=[_row_spec(), _row_spec()],
    out_shape=[jax.ShapeDtypeStruct((N, C), jnp.float32)] * 2,
)

_mid = pl.pallas_call(
    _mid_body,
    grid=(N // BL,),
    in_specs=[_pair_spec(), _row_spec(), _const_spec((C, C)),
              _const_spec((1, C)), _hist_spec()],
    out_specs=[_row_spec(), _row_spec()],
    out_shape=[jax.ShapeDtypeStruct((N, C), jnp.float32)] * 2,
)

_fin = pl.pallas_call(
    _fin_body,
    grid=(N // BL,),
    in_specs=[_pair_spec(), _row_spec(), _hist_spec()],
    out_specs=_row_spec(),
    out_shape=jax.ShapeDtypeStruct((N, C), jnp.float32),
)


def kernel(x, edge_index, W1, b1, W2, b2):
    src = edge_index[0].astype(jnp.int32)
    dst = edge_index[1].astype(jnp.int32)
    pad = E_PAD - E
    src2d = jnp.concatenate([src, jnp.zeros((pad,), jnp.int32)]).reshape(
        NW * CPW, CHUNK)
    dst2d = jnp.concatenate([dst, jnp.full((pad,), N, jnp.int32)]).reshape(
        NW * CPW, CHUNK)
    zeros_c = jnp.zeros((CHUNK, C), jnp.float32)

    hp = _deg_kernel()(dst2d).reshape(NC, N_ACC)[:, :N, None]
    g1, diag1 = _lin1(x, W1, b1.reshape(1, C), hp)
    p = _scat_kernel()(g1, src2d, dst2d, zeros_c).reshape(NC, N_ACC, C)[:, :N]
    g2, diag2 = _mid(p, diag1, W2, b2.reshape(1, C), hp)
    q = _scat_kernel()(g2, src2d, dst2d, zeros_c).reshape(NC, N_ACC, C)[:, :N]
    return _fin(q, diag2, hp)


# speedup vs baseline: 10.0875x; 1.0460x over previous
import functools

import jax
import jax.numpy as jnp
from jax import lax
from jax.experimental import pallas as pl
from jax.experimental.pallas import tpu as pltpu
from jax.experimental.pallas import tpu_sc as plsc

N = 10000
C = 128
E = 320000
NC, NS = 2, 16
NW = NC * NS
CHUNK = 128
CPW = 80
E_PAD = NW * CPW * CHUNK
N_ACC = 10240
ZR = N_ACC // NS

@functools.lru_cache(maxsize=None)
def _sc_mesh():
    return plsc.VectorSubcoreMesh(
        core_axis_name="c", subcore_axis_name="s",
        num_cores=NC, num_subcores=NS)


HROWS = N_ACC // CHUNK


def _deg_body(dst_hbm, out_hbm, idxv, hist, rowidx, shared, sem):
    c = lax.axis_index("c")
    s = lax.axis_index("s")
    wid = c * NS + s
    pltpu.sync_copy(dst_hbm.at[pl.ds(wid * CPW, CPW)], idxv)
    ones16 = jnp.ones((16,), jnp.float32)
    zeros16 = jnp.zeros((16,), jnp.float32)
    iota16 = lax.iota(jnp.int32, 16)
    for k in range(HROWS // 16):
        rowidx[pl.ds(k * 16, 16)] = iota16 + k * 16

    def zbody(r, carry):
        for k in range(8):
            hist[r, pl.ds(k * 16, 16)] = zeros16
        return carry

    lax.fori_loop(0, HROWS, zbody, 0)

    @pl.when(s == 0)
    def _():
        pltpu.sync_copy(hist, shared)
    plsc.subcore_barrier()

    def body(r, carry):
        for k in range(8):
            idx = idxv[r, pl.ds(k * 16, 16)]
            plsc.addupdate_scatter(
                hist, [lax.shift_right_logical(idx, 7),
                       lax.bitwise_and(idx, 127)], ones16)
        return carry

    lax.fori_loop(0, CPW, body, 0)
    pltpu.sync_copy(hist, shared.at[rowidx], add=True)
    plsc.subcore_barrier()

    @pl.when(s == 0)
    def _():
        pltpu.sync_copy(shared, hist)
        pltpu.sync_copy(hist, out_hbm.at[pl.ds(c * HROWS, HROWS)])


@functools.lru_cache(maxsize=None)
def _deg_kernel():
    return pl.kernel(
        _deg_body,
        out_type=jax.ShapeDtypeStruct((NC * HROWS, CHUNK), jnp.float32),
        mesh=_sc_mesh(),
        compiler_params=pltpu.CompilerParams(needs_layout_passes=False),
        scratch_types=[
            pltpu.VMEM((CPW, CHUNK), jnp.int32),
            pltpu.VMEM((HROWS, CHUNK), jnp.float32),
            pltpu.VMEM((HROWS,), jnp.int32),
            pltpu.VMEM_SHARED((HROWS, CHUNK), jnp.float32),
            pltpu.SemaphoreType.DMA,
        ],
    )


GSZ = 8
CPW0 = 112
CPW1 = 2 * CPW - CPW0


def _scat_body(g_hbm, src_hbm, dst_hbm, zeros_hbm, out_hbm,
               s0, s1, d0, d1, bufA, bufB, acc, gsA, gsB, is0, is1):
    sidx = [s0, s1]
    didx = [d0, d1]
    bufs = [bufA, bufB]
    gsems = [gsA, gsB]
    isems = [is0, is1]
    c = lax.axis_index("c")
    s = lax.axis_index("s")
    row0 = jnp.where(c == 0, s * CPW0, NS * CPW0 + s * CPW1)
    ngi = jnp.where(c == 0, CPW0 // GSZ, CPW1 // GSZ)
    pltpu.sync_copy(zeros_hbm, bufA)
    base = s * ZR
    for k in range(5):
        pltpu.sync_copy(bufA, acc.at[pl.ds(base + k * 128, 128)])
    plsc.subcore_barrier()

    pltpu.sync_copy(src_hbm.at[pl.ds(row0, GSZ)], sidx[0])
    pltpu.sync_copy(dst_hbm.at[pl.ds(row0, GSZ)], didx[0])
    pltpu.async_copy(g_hbm.at[sidx[0].at[0]], bufA, gsA)
    pltpu.async_copy(g_hbm.at[sidx[0].at[1]], bufB, gsB)
    pltpu.async_copy(src_hbm.at[pl.ds(row0 + GSZ, GSZ)], sidx[1], is1)
    pltpu.async_copy(dst_hbm.at[pl.ds(row0 + GSZ, GSZ)], didx[1], is1)

    def body(g, carry):
        p = lax.rem(g, 2)
        q = 1 - p
        for b in range(GSZ):
            buf = bufs[b % 2]
            gsem = gsems[b % 2]
            pltpu.make_async_copy(g_hbm.at[sidx[0].at[0]], buf, gsem).wait()
            for pp in range(2):
                @pl.when(p == pp)
                def _():
                    pltpu.sync_copy(buf, acc.at[didx[pp].at[b]], add=True)
            if b == GSZ - 3:
                @pl.when(g < ngi - 1)
                def _():
                    for pp in range(2):
                        @pl.when(q == pp)
                        def _():
                            pltpu.make_async_copy(
                                src_hbm.at[pl.ds(row0, GSZ)], sidx[pp],
                                isems[pp]).wait()
                            pltpu.make_async_copy(
                                dst_hbm.at[pl.ds(row0, GSZ)], didx[pp],
                                isems[pp]).wait()
            if b < GSZ - 2:
                for pp in range(2):
                    @pl.when(p == pp)
                    def _():
                        pltpu.async_copy(
                            g_hbm.at[sidx[pp].at[b + 2]], buf, gsem)
            else:
                @pl.when(g < ngi - 1)
                def _():
                    for pp in range(2):
                        @pl.when(q == pp)
                        def _():
                            pltpu.async_copy(
                                g_hbm.at[sidx[pp].at[b + 2 - GSZ]], buf, gsem)
        @pl.when(g < ngi - 2)
        def _():
            gr = row0 + (g + 2) * GSZ
            for pp in range(2):
                @pl.when(p == pp)
                def _():
                    pltpu.async_copy(src_hbm.at[pl.ds(gr, GSZ)], sidx[pp],
                                     isems[pp])
                    pltpu.async_copy(dst_hbm.at[pl.ds(gr, GSZ)], didx[pp],
                                     isems[pp])
        return carry

    lax.fori_loop(0, ngi, body, 0)
    plsc.subcore_barrier()
    for k in range(5):
        pltpu.sync_copy(acc.at[pl.ds(base + k * 128, 128)], bufA)
        pltpu.sync_copy(bufA,
                        out_hbm.at[pl.ds(c * N_ACC + base + k * 128, 128)])


@functools.lru_cache(maxsize=None)
def _scat_kernel():
    return pl.kernel(
        _scat_body,
        out_type=jax.ShapeDtypeStruct((NC * N_ACC, C), jnp.float32),
        mesh=_sc_mesh(),
        scratch_types=[
            pltpu.VMEM((GSZ, CHUNK), jnp.int32),
            pltpu.VMEM((GSZ, CHUNK), jnp.int32),
            pltpu.VMEM((GSZ, CHUNK), jnp.int32),
            pltpu.VMEM((GSZ, CHUNK), jnp.int32),
            pltpu.VMEM((CHUNK, C), jnp.float32),
            pltpu.VMEM((CHUNK, C), jnp.float32),
            pltpu.VMEM_SHARED((N_ACC, C), jnp.float32),
        ] + [pltpu.SemaphoreType.DMA] * 4,
    )


BL = 2000


def _d_from_hist(hp_ref):
    deg = hp_ref[0] + hp_ref[1] + 1.0
    return lax.rsqrt(deg)


def _lin1_body(x_ref, w_ref, b_ref, hp_ref, g_ref, diag_ref):
    d = _d_from_hist(hp_ref)
    h = jnp.dot(x_ref[...], w_ref[...], preferred_element_type=jnp.float32)
    g_ref[...] = d * h
    diag_ref[...] = (d * d) * h + b_ref[...]


def _mid_body(p_ref, diag1_ref, w_ref, b_ref, hp_ref, g_ref, diag2_ref):
    d = _d_from_hist(hp_ref)
    out1 = jnp.maximum(d * (p_ref[0] + p_ref[1]) + diag1_ref[...], 0.0)
    h2 = jnp.dot(out1, w_ref[...], preferred_element_type=jnp.float32)
    g_ref[...] = d * h2
    diag2_ref[...] = (d * d) * h2 + b_ref[...]


def _fin_body(q_ref, diag2_ref, hp_ref, o_ref):
    d = _d_from_hist(hp_ref)
    o_ref[...] = d * (q_ref[0] + q_ref[1]) + diag2_ref[...]


def _row_spec():
    return pl.BlockSpec((BL, C), lambda i: (i, 0))


def _pair_spec():
    return pl.BlockSpec((2, BL, C), lambda i: (0, i, 0))


def _hist_spec():
    return pl.BlockSpec((2, BL, 1), lambda i: (0, i, 0))


def _const_spec(shape):
    return pl.BlockSpec(shape, lambda i: (0,) * len(shape))


_lin1 = pl.pallas_call(
    _lin1_body,
    grid=(N // BL,),
    in_specs=[_row_spec(), _const_spec((C, C)), _const_spec((1, C)),
              _hist_spec()],
    out_specs=[_row_spec(), _row_spec()],
    out_shape=[jax.ShapeDtypeStruct((N, C), jnp.float32)] * 2,
)

_mid = pl.pallas_call(
    _mid_body,
    grid=(N // BL,),
    in_specs=[_pair_spec(), _row_spec(), _const_spec((C, C)),
              _const_spec((1, C)), _hist_spec()],
    out_specs=[_row_spec(), _row_spec()],
    out_shape=[jax.ShapeDtypeStruct((N, C), jnp.float32)] * 2,
)

_fin = pl.pallas_call(
    _fin_body,
    grid=(N // BL,),
    in_specs=[_pair_spec(), _row_spec(), _hist_spec()],
    out_specs=_row_spec(),
    out_shape=jax.ShapeDtypeStruct((N, C), jnp.float32),
)


def kernel(x, edge_index, W1, b1, W2, b2):
    src = edge_index[0].astype(jnp.int32)
    dst = edge_index[1].astype(jnp.int32)
    pad = E_PAD - E
    src2d = jnp.concatenate([src, jnp.zeros((pad,), jnp.int32)]).reshape(
        NW * CPW, CHUNK)
    dst2d = jnp.concatenate([dst, jnp.full((pad,), N, jnp.int32)]).reshape(
        NW * CPW, CHUNK)
    zeros_c = jnp.zeros((CHUNK, C), jnp.float32)

    hp = _deg_kernel()(dst2d).reshape(NC, N_ACC)[:, :N, None]
    g1, diag1 = _lin1(x, W1, b1.reshape(1, C), hp)
    p = _scat_kernel()(g1, src2d, dst2d, zeros_c).reshape(NC, N_ACC, C)[:, :N]
    g2, diag2 = _mid(p, diag1, W2, b2.reshape(1, C), hp)
    q = _scat_kernel()(g2, src2d, dst2d, zeros_c).reshape(NC, N_ACC, C)[:, :N]
    return _fin(q, diag2, hp)


# speedup vs baseline: 11.5581x; 1.1458x over previous
import functools

import jax
import jax.numpy as jnp
from jax import lax
from jax.experimental import pallas as pl
from jax.experimental.pallas import tpu as pltpu
from jax.experimental.pallas import tpu_sc as plsc

N = 10000
C = 128
E = 320000
NC, NS = 2, 16
NW = NC * NS
CHUNK = 128
CPW = 80
E_PAD = NW * CPW * CHUNK
N_ACC = 10240
ZR = N_ACC // NS

@functools.lru_cache(maxsize=None)
def _sc_mesh():
    return plsc.VectorSubcoreMesh(
        core_axis_name="c", subcore_axis_name="s",
        num_cores=NC, num_subcores=NS)


HROWS = N_ACC // CHUNK


def _deg_body(dst_hbm, out_hbm, idxv, hist, rowidx, shared, sem):
    c = lax.axis_index("c")
    s = lax.axis_index("s")
    wid = c * NS + s
    pltpu.sync_copy(dst_hbm.at[pl.ds(wid * CPW, CPW)], idxv)
    ones16 = jnp.ones((16,), jnp.float32)
    zeros16 = jnp.zeros((16,), jnp.float32)
    iota16 = lax.iota(jnp.int32, 16)
    for k in range(HROWS // 16):
        rowidx[pl.ds(k * 16, 16)] = iota16 + k * 16

    def zbody(r, carry):
        for k in range(8):
            hist[r, pl.ds(k * 16, 16)] = zeros16
        return carry

    lax.fori_loop(0, HROWS, zbody, 0)

    @pl.when(s == 0)
    def _():
        pltpu.sync_copy(hist, shared)
    plsc.subcore_barrier()

    def body(r, carry):
        for k in range(8):
            idx = idxv[r, pl.ds(k * 16, 16)]
            plsc.addupdate_scatter(
                hist, [lax.shift_right_logical(idx, 7),
                       lax.bitwise_and(idx, 127)], ones16)
        return carry

    lax.fori_loop(0, CPW, body, 0)
    pltpu.sync_copy(hist, shared.at[rowidx], add=True)
    plsc.subcore_barrier()

    @pl.when(s == 0)
    def _():
        pltpu.sync_copy(shared, hist)
        pltpu.sync_copy(hist, out_hbm.at[pl.ds(c * HROWS, HROWS)])


@functools.lru_cache(maxsize=None)
def _deg_kernel():
    return pl.kernel(
        _deg_body,
        out_type=jax.ShapeDtypeStruct((NC * HROWS, CHUNK), jnp.float32),
        mesh=_sc_mesh(),
        compiler_params=pltpu.CompilerParams(needs_layout_passes=False),
        scratch_types=[
            pltpu.VMEM((CPW, CHUNK), jnp.int32),
            pltpu.VMEM((HROWS, CHUNK), jnp.float32),
            pltpu.VMEM((HROWS,), jnp.int32),
            pltpu.VMEM_SHARED((HROWS, CHUNK), jnp.float32),
            pltpu.SemaphoreType.DMA,
        ],
    )


GSZ = 8
CPW0 = 144
CPW1 = 2 * CPW - CPW0


def _scat_body(g_hbm, src_hbm, dst_hbm, zeros_hbm, out_hbm,
               s0, s1, d0, d1, bufA, bufB, acc, gsA, gsB, is0, is1):
    sidx = [s0, s1]
    didx = [d0, d1]
    bufs = [bufA, bufB]
    gsems = [gsA, gsB]
    isems = [is0, is1]
    c = lax.axis_index("c")
    s = lax.axis_index("s")
    row0 = jnp.where(c == 0, s * CPW0, NS * CPW0 + s * CPW1)
    ngi = jnp.where(c == 0, CPW0 // GSZ, CPW1 // GSZ)
    pltpu.sync_copy(zeros_hbm, bufA)
    base = s * ZR
    for k in range(5):
        pltpu.sync_copy(bufA, acc.at[pl.ds(base + k * 128, 128)])
    plsc.subcore_barrier()

    pltpu.sync_copy(src_hbm.at[pl.ds(row0, GSZ)], sidx[0])
    pltpu.sync_copy(dst_hbm.at[pl.ds(row0, GSZ)], didx[0])
    pltpu.async_copy(g_hbm.at[sidx[0].at[0]], bufA, gsA)
    pltpu.async_copy(g_hbm.at[sidx[0].at[1]], bufB, gsB)
    pltpu.async_copy(src_hbm.at[pl.ds(row0 + GSZ, GSZ)], sidx[1], is1)
    pltpu.async_copy(dst_hbm.at[pl.ds(row0 + GSZ, GSZ)], didx[1], is1)

    def body(g, carry):
        p = lax.rem(g, 2)
        q = 1 - p
        for b in range(GSZ):
            buf = bufs[b % 2]
            gsem = gsems[b % 2]
            pltpu.make_async_copy(g_hbm.at[sidx[0].at[0]], buf, gsem).wait()
            for pp in range(2):
                @pl.when(p == pp)
                def _():
                    pltpu.sync_copy(buf, acc.at[didx[pp].at[b]], add=True)
            if b == GSZ - 3:
                @pl.when(g < ngi - 1)
                def _():
                    for pp in range(2):
                        @pl.when(q == pp)
                        def _():
                            pltpu.make_async_copy(
                                src_hbm.at[pl.ds(row0, GSZ)], sidx[pp],
                                isems[pp]).wait()
                            pltpu.make_async_copy(
                                dst_hbm.at[pl.ds(row0, GSZ)], didx[pp],
                                isems[pp]).wait()
            if b < GSZ - 2:
                for pp in range(2):
                    @pl.when(p == pp)
                    def _():
                        pltpu.async_copy(
                            g_hbm.at[sidx[pp].at[b + 2]], buf, gsem)
            else:
                @pl.when(g < ngi - 1)
                def _():
                    for pp in range(2):
                        @pl.when(q == pp)
                        def _():
                            pltpu.async_copy(
                                g_hbm.at[sidx[pp].at[b + 2 - GSZ]], buf, gsem)
        @pl.when(g < ngi - 2)
        def _():
            gr = row0 + (g + 2) * GSZ
            for pp in range(2):
                @pl.when(p == pp)
                def _():
                    pltpu.async_copy(src_hbm.at[pl.ds(gr, GSZ)], sidx[pp],
                                     isems[pp])
                    pltpu.async_copy(dst_hbm.at[pl.ds(gr, GSZ)], didx[pp],
                                     isems[pp])
        return carry

    lax.fori_loop(0, ngi, body, 0)
    plsc.subcore_barrier()
    for k in range(5):
        pltpu.sync_copy(acc.at[pl.ds(base + k * 128, 128)], bufA)
        pltpu.sync_copy(bufA,
                        out_hbm.at[pl.ds(c * N_ACC + base + k * 128, 128)])


@functools.lru_cache(maxsize=None)
def _scat_kernel():
    return pl.kernel(
        _scat_body,
        out_type=jax.ShapeDtypeStruct((NC * N_ACC, C), jnp.float32),
        mesh=_sc_mesh(),
        scratch_types=[
            pltpu.VMEM((GSZ, CHUNK), jnp.int32),
            pltpu.VMEM((GSZ, CHUNK), jnp.int32),
            pltpu.VMEM((GSZ, CHUNK), jnp.int32),
            pltpu.VMEM((GSZ, CHUNK), jnp.int32),
            pltpu.VMEM((CHUNK, C), jnp.float32),
            pltpu.VMEM((CHUNK, C), jnp.float32),
            pltpu.VMEM_SHARED((N_ACC, C), jnp.float32),
        ] + [pltpu.SemaphoreType.DMA] * 4,
    )


BL = 2000


def _d_from_hist(hp_ref):
    deg = hp_ref[0] + hp_ref[1] + 1.0
    return lax.rsqrt(deg)


def _lin1_body(x_ref, w_ref, b_ref, hp_ref, g_ref, diag_ref):
    d = _d_from_hist(hp_ref)
    h = jnp.dot(x_ref[...], w_ref[...], preferred_element_type=jnp.float32)
    g_ref[...] = d * h
    diag_ref[...] = (d * d) * h + b_ref[...]


def _mid_body(p_ref, diag1_ref, w_ref, b_ref, hp_ref, g_ref, diag2_ref):
    d = _d_from_hist(hp_ref)
    out1 = jnp.maximum(d * (p_ref[0] + p_ref[1]) + diag1_ref[...], 0.0)
    h2 = jnp.dot(out1, w_ref[...], preferred_element_type=jnp.float32)
    g_ref[...] = d * h2
    diag2_ref[...] = (d * d) * h2 + b_ref[...]


def _fin_body(q_ref, diag2_ref, hp_ref, o_ref):
    d = _d_from_hist(hp_ref)
    o_ref[...] = d * (q_ref[0] + q_ref[1]) + diag2_ref[...]


def _row_spec():
    return pl.BlockSpec((BL, C), lambda i: (i, 0))


def _pair_spec():
    return pl.BlockSpec((2, BL, C), lambda i: (0, i, 0))


def _hist_spec():
    return pl.BlockSpec((2, BL, 1), lambda i: (0, i, 0))


def _const_spec(shape):
    return pl.BlockSpec(shape, lambda i: (0,) * len(shape))


_lin1 = pl.pallas_call(
    _lin1_body,
    grid=(N // BL,),
    in_specs=[_row_spec(), _const_spec((C, C)), _const_spec((1, C)),
              _hist_spec()],
    out_specs=[_row_spec(), _row_spec()],
    out_shape=[jax.ShapeDtypeStruct((N, C), jnp.float32)] * 2,
)

_mid = pl.pallas_call(
    _mid_body,
    grid=(N // BL,),
    in_specs=[_pair_spec(), _row_spec(), _const_spec((C, C)),
              _const_spec((1, C)), _hist_spec()],
    out_specs=[_row_spec(), _row_spec()],
    out_shape=[jax.ShapeDtypeStruct((N, C), jnp.float32)] * 2,
)

_fin = pl.pallas_call(
    _fin_body,
    grid=(N // BL,),
    in_specs=[_pair_spec(), _row_spec(), _hist_spec()],
    out_specs=_row_spec(),
    out_shape=jax.ShapeDtypeStruct((N, C), jnp.float32),
)


def kernel(x, edge_index, W1, b1, W2, b2):
    src = edge_index[0].astype(jnp.int32)
    dst = edge_index[1].astype(jnp.int32)
    pad = E_PAD - E
    src2d = jnp.concatenate([src, jnp.zeros((pad,), jnp.int32)]).reshape(
        NW * CPW, CHUNK)
    dst2d = jnp.concatenate([dst, jnp.full((pad,), N, jnp.int32)]).reshape(
        NW * CPW, CHUNK)
    zeros_c = jnp.zeros((CHUNK, C), jnp.float32)

    hp = _deg_kernel()(dst2d).reshape(NC, N_ACC)[:, :N, None]
    g1, diag1 = _lin1(x, W1, b1.reshape(1, C), hp)
    p = _scat_kernel()(g1, src2d, dst2d, zeros_c).reshape(NC, N_ACC, C)[:, :N]
    g2, diag2 = _mid(p, diag1, W2, b2.reshape(1, C), hp)
    q = _scat_kernel()(g2, src2d, dst2d, zeros_c).reshape(NC, N_ACC, C)[:, :N]
    return _fin(q, diag2, hp)


# speedup vs baseline: 11.6710x; 1.0098x over previous
import functools

import jax
import jax.numpy as jnp
from jax import lax
from jax.experimental import pallas as pl
from jax.experimental.pallas import tpu as pltpu
from jax.experimental.pallas import tpu_sc as plsc

N = 10000
C = 128
E = 320000
NC, NS = 2, 16
NW = NC * NS
CHUNK = 128
CPW = 80
E_PAD = NW * CPW * CHUNK
N_ACC = 10240
ZR = N_ACC // NS

@functools.lru_cache(maxsize=None)
def _sc_mesh():
    return plsc.VectorSubcoreMesh(
        core_axis_name="c", subcore_axis_name="s",
        num_cores=NC, num_subcores=NS)


HROWS = N_ACC // CHUNK


def _deg_body(dst_hbm, out_hbm, idxv, hist, rowidx, shared, sem):
    c = lax.axis_index("c")
    s = lax.axis_index("s")
    wid = c * NS + s
    pltpu.sync_copy(dst_hbm.at[pl.ds(wid * CPW, CPW)], idxv)
    ones16 = jnp.ones((16,), jnp.float32)
    zeros16 = jnp.zeros((16,), jnp.float32)
    iota16 = lax.iota(jnp.int32, 16)
    for k in range(HROWS // 16):
        rowidx[pl.ds(k * 16, 16)] = iota16 + k * 16

    def zbody(r, carry):
        for k in range(8):
            hist[r, pl.ds(k * 16, 16)] = zeros16
        return carry

    lax.fori_loop(0, HROWS, zbody, 0)

    @pl.when(s == 0)
    def _():
        pltpu.sync_copy(hist, shared)
    plsc.subcore_barrier()

    def body(r, carry):
        for k in range(8):
            idx = idxv[r, pl.ds(k * 16, 16)]
            plsc.addupdate_scatter(
                hist, [lax.shift_right_logical(idx, 7),
                       lax.bitwise_and(idx, 127)], ones16)
        return carry

    lax.fori_loop(0, CPW, body, 0)
    pltpu.sync_copy(hist, shared.at[rowidx], add=True)
    plsc.subcore_barrier()

    @pl.when(s == 0)
    def _():
        pltpu.sync_copy(shared, hist)
        pltpu.sync_copy(hist, out_hbm.at[pl.ds(c * HROWS, HROWS)])


@functools.lru_cache(maxsize=None)
def _deg_kernel():
    return pl.kernel(
        _deg_body,
        out_type=jax.ShapeDtypeStruct((NC * HROWS, CHUNK), jnp.float32),
        mesh=_sc_mesh(),
        compiler_params=pltpu.CompilerParams(needs_layout_passes=False),
        scratch_types=[
            pltpu.VMEM((CPW, CHUNK), jnp.int32),
            pltpu.VMEM((HROWS, CHUNK), jnp.float32),
            pltpu.VMEM((HROWS,), jnp.int32),
            pltpu.VMEM_SHARED((HROWS, CHUNK), jnp.float32),
            pltpu.SemaphoreType.DMA,
        ],
    )


GSZ = 8
CPW0 = 152
CPW1 = 2 * CPW - CPW0


def _scat_body(g_hbm, src_hbm, dst_hbm, zeros_hbm, out_hbm,
               s0, s1, d0, d1, bufA, bufB, acc, gsA, gsB, is0, is1):
    sidx = [s0, s1]
    didx = [d0, d1]
    bufs = [bufA, bufB]
    gsems = [gsA, gsB]
    isems = [is0, is1]
    c = lax.axis_index("c")
    s = lax.axis_index("s")
    row0 = jnp.where(c == 0, s * CPW0, NS * CPW0 + s * CPW1)
    ngi = jnp.where(c == 0, CPW0 // GSZ, CPW1 // GSZ)
    pltpu.sync_copy(zeros_hbm, bufA)
    base = s * ZR
    for k in range(5):
        pltpu.sync_copy(bufA, acc.at[pl.ds(base + k * 128, 128)])
    plsc.subcore_barrier()

    pltpu.sync_copy(src_hbm.at[pl.ds(row0, GSZ)], sidx[0])
    pltpu.sync_copy(dst_hbm.at[pl.ds(row0, GSZ)], didx[0])
    pltpu.async_copy(g_hbm.at[sidx[0].at[0]], bufA, gsA)
    pltpu.async_copy(g_hbm.at[sidx[0].at[1]], bufB, gsB)

    @pl.when(ngi > 1)
    def _():
        pltpu.async_copy(src_hbm.at[pl.ds(row0 + GSZ, GSZ)], sidx[1], is1)
        pltpu.async_copy(dst_hbm.at[pl.ds(row0 + GSZ, GSZ)], didx[1], is1)

    def body(g, carry):
        p = lax.rem(g, 2)
        q = 1 - p
        for b in range(GSZ):
            buf = bufs[b % 2]
            gsem = gsems[b % 2]
            pltpu.make_async_copy(g_hbm.at[sidx[0].at[0]], buf, gsem).wait()
            for pp in range(2):
                @pl.when(p == pp)
                def _():
                    pltpu.sync_copy(buf, acc.at[didx[pp].at[b]], add=True)
            if b == GSZ - 3:
                @pl.when(g < ngi - 1)
                def _():
                    for pp in range(2):
                        @pl.when(q == pp)
                        def _():
                            pltpu.make_async_copy(
                                src_hbm.at[pl.ds(row0, GSZ)], sidx[pp],
                                isems[pp]).wait()
                            pltpu.make_async_copy(
                                dst_hbm.at[pl.ds(row0, GSZ)], didx[pp],
                                isems[pp]).wait()
            if b < GSZ - 2:
                for pp in range(2):
                    @pl.when(p == pp)
                    def _():
                        pltpu.async_copy(
                            g_hbm.at[sidx[pp].at[b + 2]], buf, gsem)
            else:
                @pl.when(g < ngi - 1)
                def _():
                    for pp in range(2):
                        @pl.when(q == pp)
                        def _():
                            pltpu.async_copy(
                                g_hbm.at[sidx[pp].at[b + 2 - GSZ]], buf, gsem)
        @pl.when(g < ngi - 2)
        def _():
            gr = row0 + (g + 2) * GSZ
            for pp in range(2):
                @pl.when(p == pp)
                def _():
                    pltpu.async_copy(src_hbm.at[pl.ds(gr, GSZ)], sidx[pp],
                                     isems[pp])
                    pltpu.async_copy(dst_hbm.at[pl.ds(gr, GSZ)], didx[pp],
                                     isems[pp])
        return carry

    lax.fori_loop(0, ngi, body, 0)
    plsc.subcore_barrier()
    for k in range(5):
        pltpu.sync_copy(acc.at[pl.ds(base + k * 128, 128)], bufA)
        pltpu.sync_copy(bufA,
                        out_hbm.at[pl.ds(c * N_ACC + base + k * 128, 128)])


@functools.lru_cache(maxsize=None)
def _scat_kernel():
    return pl.kernel(
        _scat_body,
        out_type=jax.ShapeDtypeStruct((NC * N_ACC, C), jnp.float32),
        mesh=_sc_mesh(),
        scratch_types=[
            pltpu.VMEM((GSZ, CHUNK), jnp.int32),
            pltpu.VMEM((GSZ, CHUNK), jnp.int32),
            pltpu.VMEM((GSZ, CHUNK), jnp.int32),
            pltpu.VMEM((GSZ, CHUNK), jnp.int32),
            pltpu.VMEM((CHUNK, C), jnp.float32),
            pltpu.VMEM((CHUNK, C), jnp.float32),
            pltpu.VMEM_SHARED((N_ACC, C), jnp.float32),
        ] + [pltpu.SemaphoreType.DMA] * 4,
    )


BL = 2000


def _d_from_hist(hp_ref):
    deg = hp_ref[0] + hp_ref[1] + 1.0
    return lax.rsqrt(deg)


def _lin1_body(x_ref, w_ref, b_ref, hp_ref, g_ref, diag_ref):
    d = _d_from_hist(hp_ref)
    h = jnp.dot(x_ref[...], w_ref[...], preferred_element_type=jnp.float32)
    g_ref[...] = d * h
    diag_ref[...] = (d * d) * h + b_ref[...]


def _mid_body(p_ref, diag1_ref, w_ref, b_ref, hp_ref, g_ref, diag2_ref):
    d = _d_from_hist(hp_ref)
    out1 = jnp.maximum(d * (p_ref[0] + p_ref[1]) + diag1_ref[...], 0.0)
    h2 = jnp.dot(out1, w_ref[...], preferred_element_type=jnp.float32)
    g_ref[...] = d * h2
    diag2_ref[...] = (d * d) * h2 + b_ref[...]


def _fin_body(q_ref, diag2_ref, hp_ref, o_ref):
    d = _d_from_hist(hp_ref)
    o_ref[...] = d * (q_ref[0] + q_ref[1]) + diag2_ref[...]


def _row_spec():
    return pl.BlockSpec((BL, C), lambda i: (i, 0))


def _pair_spec():
    return pl.BlockSpec((2, BL, C), lambda i: (0, i, 0))


def _hist_spec():
    return pl.BlockSpec((2, BL, 1), lambda i: (0, i, 0))


def _const_spec(shape):
    return pl.BlockSpec(shape, lambda i: (0,) * len(shape))


_lin1 = pl.pallas_call(
    _lin1_body,
    grid=(N // BL,),
    in_specs=[_row_spec(), _const_spec((C, C)), _const_spec((1, C)),
              _hist_spec()],
    out_specs=[_row_spec(), _row_spec()],
    out_shape=[jax.ShapeDtypeStruct((N, C), jnp.float32)] * 2,
)

_mid = pl.pallas_call(
    _mid_body,
    grid=(N // BL,),
    in_specs=[_pair_spec(), _row_spec(), _const_spec((C, C)),
              _const_spec((1, C)), _hist_spec()],
    out_specs=[_row_spec(), _row_spec()],
    out_shape=[jax.ShapeDtypeStruct((N, C), jnp.float32)] * 2,
)

_fin = pl.pallas_call(
    _fin_body,
    grid=(N // BL,),
    in_specs=[_pair_spec(), _row_spec(), _hist_spec()],
    out_specs=_row_spec(),
    out_shape=jax.ShapeDtypeStruct((N, C), jnp.float32),
)


def kernel(x, edge_index, W1, b1, W2, b2):
    src = edge_index[0].astype(jnp.int32)
    dst = edge_index[1].astype(jnp.int32)
    pad = E_PAD - E
    src2d = jnp.concatenate([src, jnp.zeros((pad,), jnp.int32)]).reshape(
        NW * CPW, CHUNK)
    dst2d = jnp.concatenate([dst, jnp.full((pad,), N, jnp.int32)]).reshape(
        NW * CPW, CHUNK)
    zeros_c = jnp.zeros((CHUNK, C), jnp.float32)

    hp = _deg_kernel()(dst2d).reshape(NC, N_ACC)[:, :N, None]
    g1, diag1 = _lin1(x, W1, b1.reshape(1, C), hp)
    p = _scat_kernel()(g1, src2d, dst2d, zeros_c).reshape(NC, N_ACC, C)[:, :N]
    g2, diag2 = _mid(p, diag1, W2, b2.reshape(1, C), hp)
    q = _scat_kernel()(g2, src2d, dst2d, zeros_c).reshape(NC, N_ACC, C)[:, :N]
    return _fin(q, diag2, hp)


# speedup vs baseline: 11.7245x; 1.0046x over previous
import functools

import jax
import jax.numpy as jnp
from jax import lax
from jax.experimental import pallas as pl
from jax.experimental.pallas import tpu as pltpu
from jax.experimental.pallas import tpu_sc as plsc

N = 10000
C = 128
E = 320000
NC, NS = 2, 16
NW = NC * NS
CHUNK = 128
CPW = 80
E_PAD = NW * CPW * CHUNK
N_ACC = 10240
ZR = N_ACC // NS

@functools.lru_cache(maxsize=None)
def _sc_mesh():
    return plsc.VectorSubcoreMesh(
        core_axis_name="c", subcore_axis_name="s",
        num_cores=NC, num_subcores=NS)


HROWS = N_ACC // CHUNK


def _deg_body(dst_hbm, out_hbm, idxv, hist, rowidx, shared, sem):
    c = lax.axis_index("c")
    s = lax.axis_index("s")
    wid = c * NS + s
    pltpu.sync_copy(dst_hbm.at[pl.ds(wid * CPW, CPW)], idxv)
    ones16 = jnp.ones((16,), jnp.float32)
    zeros16 = jnp.zeros((16,), jnp.float32)
    iota16 = lax.iota(jnp.int32, 16)
    for k in range(HROWS // 16):
        rowidx[pl.ds(k * 16, 16)] = iota16 + k * 16

    def zbody(r, carry):
        for k in range(8):
            hist[r, pl.ds(k * 16, 16)] = zeros16
        return carry

    lax.fori_loop(0, HROWS, zbody, 0)

    @pl.when(s == 0)
    def _():
        pltpu.sync_copy(hist, shared)
    plsc.subcore_barrier()

    def body(r, carry):
        for k in range(8):
            idx = idxv[r, pl.ds(k * 16, 16)]
            plsc.addupdate_scatter(
                hist, [lax.shift_right_logical(idx, 7),
                       lax.bitwise_and(idx, 127)], ones16)
        return carry

    lax.fori_loop(0, CPW, body, 0)
    pltpu.sync_copy(hist, shared.at[rowidx], add=True)
    plsc.subcore_barrier()

    @pl.when(s == 0)
    def _():
        pltpu.sync_copy(shared, hist)
        pltpu.sync_copy(hist, out_hbm.at[pl.ds(c * HROWS, HROWS)])


@functools.lru_cache(maxsize=None)
def _deg_kernel():
    return pl.kernel(
        _deg_body,
        out_type=jax.ShapeDtypeStruct((NC * HROWS, CHUNK), jnp.float32),
        mesh=_sc_mesh(),
        compiler_params=pltpu.CompilerParams(needs_layout_passes=False),
        scratch_types=[
            pltpu.VMEM((CPW, CHUNK), jnp.int32),
            pltpu.VMEM((HROWS, CHUNK), jnp.float32),
            pltpu.VMEM((HROWS,), jnp.int32),
            pltpu.VMEM_SHARED((HROWS, CHUNK), jnp.float32),
            pltpu.SemaphoreType.DMA,
        ],
    )


GSZ = 8
CPW0 = 152
CPW1 = 2 * CPW - CPW0


def _scat_body(g_hbm, src_hbm, dst_hbm, zeros_hbm, out_hbm,
               s0, s1, d0, d1, bufA, bufB, acc, gsA, gsB, is0, is1):
    sidx = [s0, s1]
    didx = [d0, d1]
    bufs = [bufA, bufB]
    gsems = [gsA, gsB]
    isems = [is0, is1]
    c = lax.axis_index("c")
    s = lax.axis_index("s")
    row0 = jnp.where(c == 0, s * CPW0, NS * CPW0 + s * CPW1)
    ngi = jnp.where(c == 0, CPW0 // GSZ, CPW1 // GSZ)
    pltpu.sync_copy(zeros_hbm, bufA)
    base = s * ZR
    for k in range(5):
        pltpu.sync_copy(bufA, acc.at[pl.ds(base + k * 128, 128)])
    plsc.subcore_barrier()

    pltpu.sync_copy(src_hbm.at[pl.ds(row0, GSZ)], sidx[0])
    pltpu.sync_copy(dst_hbm.at[pl.ds(row0, GSZ)], didx[0])
    pltpu.async_copy(g_hbm.at[sidx[0].at[0]], bufA, gsA)
    pltpu.async_copy(g_hbm.at[sidx[0].at[1]], bufB, gsB)

    @pl.when(ngi > 1)
    def _():
        pltpu.async_copy(src_hbm.at[pl.ds(row0 + GSZ, GSZ)], sidx[1], is1)
        pltpu.async_copy(dst_hbm.at[pl.ds(row0 + GSZ, GSZ)], didx[1], is1)

    def body(g, carry):
        p = lax.rem(g, 2)
        q = 1 - p
        for b in range(GSZ):
            buf = bufs[b % 2]
            gsem = gsems[b % 2]
            pltpu.make_async_copy(g_hbm.at[sidx[0].at[0]], buf, gsem).wait()
            for pp in range(2):
                @pl.when(p == pp)
                def _():
                    pltpu.sync_copy(buf, acc.at[didx[pp].at[b]], add=True)
            if b == GSZ - 3:
                @pl.when(g < ngi - 1)
                def _():
                    for pp in range(2):
                        @pl.when(q == pp)
                        def _():
                            pltpu.make_async_copy(
                                src_hbm.at[pl.ds(row0, GSZ)], sidx[pp],
                                isems[pp]).wait()
                            pltpu.make_async_copy(
                                dst_hbm.at[pl.ds(row0, GSZ)], didx[pp],
                                isems[pp]).wait()
            if b < GSZ - 2:
                for pp in range(2):
                    @pl.when(p == pp)
                    def _():
                        pltpu.async_copy(
                            g_hbm.at[sidx[pp].at[b + 2]], buf, gsem)
            else:
                @pl.when(g < ngi - 1)
                def _():
                    for pp in range(2):
                        @pl.when(q == pp)
                        def _():
                            pltpu.async_copy(
                                g_hbm.at[sidx[pp].at[b + 2 - GSZ]], buf, gsem)
        @pl.when(g < ngi - 2)
        def _():
            gr = row0 + (g + 2) * GSZ
            for pp in range(2):
                @pl.when(p == pp)
                def _():
                    pltpu.async_copy(src_hbm.at[pl.ds(gr, GSZ)], sidx[pp],
                                     isems[pp])
                    pltpu.async_copy(dst_hbm.at[pl.ds(gr, GSZ)], didx[pp],
                                     isems[pp])
        return carry

    lax.fori_loop(0, ngi, body, 0)
    plsc.subcore_barrier()
    for k in range(5):
        pltpu.sync_copy(acc.at[pl.ds(base + k * 128, 128)], bufA)
        pltpu.sync_copy(bufA,
                        out_hbm.at[pl.ds(c * N_ACC + base + k * 128, 128)])


@functools.lru_cache(maxsize=None)
def _scat_kernel():
    return pl.kernel(
        _scat_body,
        out_type=jax.ShapeDtypeStruct((NC * N_ACC, C), jnp.float32),
        mesh=_sc_mesh(),
        scratch_types=[
            pltpu.VMEM((GSZ, CHUNK), jnp.int32),
            pltpu.VMEM((GSZ, CHUNK), jnp.int32),
            pltpu.VMEM((GSZ, CHUNK), jnp.int32),
            pltpu.VMEM((GSZ, CHUNK), jnp.int32),
            pltpu.VMEM((CHUNK, C), jnp.float32),
            pltpu.VMEM((CHUNK, C), jnp.float32),
            pltpu.VMEM_SHARED((N_ACC, C), jnp.float32),
        ] + [pltpu.SemaphoreType.DMA] * 4,
    )


BL = 2000


def _d_from_hist(hp_ref):
    deg = hp_ref[0] + hp_ref[1] + 1.0
    return lax.rsqrt(deg)


def _lin1_body(x_ref, w_ref, b_ref, hp_ref, g_ref, diag_ref):
    d = _d_from_hist(hp_ref)
    h = jnp.dot(x_ref[...], w_ref[...], preferred_element_type=jnp.float32)
    g_ref[...] = d * h
    diag_ref[...] = (d * d) * h + b_ref[...]


def _mid_body(p_ref, diag1_ref, w_ref, b_ref, hp_ref, g_ref, diag2_ref):
    d = _d_from_hist(hp_ref)
    out1 = jnp.maximum(d * (p_ref[0] + p_ref[1]) + diag1_ref[...], 0.0)
    h2 = jnp.dot(out1, w_ref[...], preferred_element_type=jnp.float32)
    g_ref[...] = d * h2
    diag2_ref[...] = (d * d) * h2 + b_ref[...]


def _fin_body(q_ref, diag2_ref, hp_ref, o_ref):
    d = _d_from_hist(hp_ref)
    o_ref[...] = d * (q_ref[0] + q_ref[1]) + diag2_ref[...]


def _row_spec():
    return pl.BlockSpec((BL, C), lambda i: (i, 0))


def _pair_spec():
    return pl.BlockSpec((2, BL, C), lambda i: (0, i, 0))


def _hist_spec():
    return pl.BlockSpec((2, BL, 1), lambda i: (0, i, 0))


def _const_spec(shape):
    return pl.BlockSpec(shape, lambda i: (0,) * len(shape))


_lin1 = pl.pallas_call(
    _lin1_body,
    grid=(N // BL,),
    in_specs=[_row_spec(), _const_spec((C, C)), _const_spec((1, C)),
              _hist_spec()],
    out_specs=[_row_spec(), _row_spec()],
    out_shape=[jax.ShapeDtypeStruct((N, C), jnp.float32)] * 2,
)

_mid = pl.pallas_call(
    _mid_body,
    grid=(N // BL,),
    in_specs=[_pair_spec(), _row_spec(), _const_spec((C, C)),
              _const_spec((1, C)), _hist_spec()],
    out_specs=[_row_spec(), _row_spec()],
    out_shape=[jax.ShapeDtypeStruct((N, C), jnp.float32)] * 2,
)

_fin = pl.pallas_call(
    _fin_body,
    grid=(N // BL,),
    in_specs=[_pair_spec(), _row_spec(), _hist_spec()],
    out_specs=_row_spec(),
    out_shape=jax.ShapeDtypeStruct((N, C), jnp.float32),
)


def kernel(x, edge_index, W1, b1, W2, b2):
    src = edge_index[0].astype(jnp.int32)
    dst = edge_index[1].astype(jnp.int32)
    pad = E_PAD - E
    src2d = jnp.concatenate([src, jnp.zeros((pad,), jnp.int32)]).reshape(
        NW * CPW, CHUNK)
    dst2d = jnp.concatenate([dst, jnp.full((pad,), N, jnp.int32)]).reshape(
        NW * CPW, CHUNK)
    zeros_c = jnp.zeros((CHUNK, C), jnp.float32)

    hp = _deg_kernel()(dst2d).reshape(NC, N_ACC, 1)
    g1, diag1 = _lin1(x, W1, b1.reshape(1, C), hp)
    p = _scat_kernel()(g1, src2d, dst2d, zeros_c).reshape(NC, N_ACC, C)
    g2, diag2 = _mid(p, diag1, W2, b2.reshape(1, C), hp)
    q = _scat_kernel()(g2, src2d, dst2d, zeros_c).reshape(NC, N_ACC, C)
    return _fin(q, diag2, hp)
